# baseline (speedup 1.0000x reference)
import functools

import jax
import jax.numpy as jnp
from jax import lax
from jax.experimental import pallas as pl
from jax.experimental.pallas import tpu as pltpu
from jax.experimental.pallas import tpu_sc as plsc

BATCH = 4096
CODEBOOK = 1000000
NUM_LOOKUPS = 1024
EMB = 32
JC = 128
NJC = NUM_LOOKUPS // JC
RB = 8
NC, NS = 2, 16
NW = NC * NS
ROWS_PER_W = BATCH // NW
NGRP = ROWS_PER_W // RB
NSTEP = NGRP * NJC
UNROLL = 8


TCH = 512
TGRID = CODEBOOK // (4 * TCH)
TMAIN = TGRID * 4 * TCH
TTAIL = CODEBOOK - TMAIN
TTCH = TTAIL // 4
TROWS = CODEBOOK * EMB // 128


def _tc_relayout(t4):
    def main_body(in0, in1, in2, in3, out_ref):
        xs = jnp.concatenate(
            [in0[...], in1[...], in2[...], in3[...]], axis=0)
        eye = jnp.eye(128, dtype=jnp.float32)
        out_ref[...] = lax.dot_general(
            xs, eye, (((0,), (0,)), ((), ())),
            precision=lax.Precision.HIGHEST)

    in_specs = [
        pl.BlockSpec((32, TCH), lambda i, a=a: (0, 4 * i + a))
        for a in range(4)
    ]
    t2 = pl.pallas_call(
        main_body,
        grid=(TGRID,),
        in_specs=in_specs,
        out_specs=pl.BlockSpec((TCH, 128), lambda i: (i, 0)),
        out_shape=jax.ShapeDtypeStruct((TROWS, 128), jnp.float32),
    )(t4, t4, t4, t4)

    t4_tail = lax.slice(t4, (0, TMAIN), (32, CODEBOOK))
    ytail = (t4_tail.reshape(32, 4, TTCH).transpose(1, 0, 2)
             .reshape(128, TTCH).T)
    return lax.dynamic_update_slice(t2, ytail, (TMAIN * EMB // 128, 0))


def _sc_logits(codes_t, table, w2):
    mesh = plsc.VectorSubcoreMesh(core_axis_name="c", subcore_axis_name="s")

    @functools.partial(
        pl.kernel,
        out_type=jax.ShapeDtypeStruct((BATCH,), jnp.float32),
        mesh=mesh,
        compiler_params=pltpu.CompilerParams(
            needs_layout_passes=False, use_tc_tiling_on_sc=False),
        scratch_types=[
            pltpu.VMEM((2, JC, RB), jnp.int32),
            pltpu.VMEM((2, RB, JC), jnp.int32),
            pltpu.VMEM((2, RB, JC, EMB), jnp.float32),
            pltpu.VMEM((NUM_LOOKUPS * EMB // 128, 128), jnp.float32),
            pltpu.VMEM((16 * ROWS_PER_W,), jnp.float32),
            pltpu.VMEM((ROWS_PER_W,), jnp.float32),
            pltpu.SemaphoreType.DMA,
            pltpu.SemaphoreType.DMA,
            pltpu.SemaphoreType.DMA,
            pltpu.SemaphoreType.DMA,
        ],
    )
    def run(codes_hbm, table_hbm, w_hbm, out_hbm,
            blk_v, idx_v, rows_v, w_v, acc_t_v, out_v,
            semc0, semc1, semg0, semg1):
        wid = lax.axis_index("s") * NC + lax.axis_index("c")
        base = wid * ROWS_PER_W
        semc = (semc0, semc1)
        semg = (semg0, semg1)
        lane = lax.iota(jnp.int32, 16)

        pltpu.sync_copy(w_hbm, w_v)

        def blk_src(grp, jc):
            col = base + grp * RB
            return codes_hbm.at[pl.ds(jc * JC, JC), pl.ds(col, RB)]

        def fire_blk(grp, jc, slot):
            pltpu.make_async_copy(blk_src(grp, jc), blk_v.at[slot],
                                  semc[slot]).start()

        def wait_blk(grp, jc, slot):
            pltpu.make_async_copy(blk_src(grp, jc), blk_v.at[slot],
                                  semc[slot]).wait()

        def transpose_blk(slot):
            for r in range(RB):
                rvec = jnp.full((16,), r, jnp.int32)
                for j0 in range(0, JC, 16):
                    v = plsc.load_gather(blk_v.at[slot], [j0 + lane, rvec])
                    idx_v[slot, r, pl.ds(j0, 16)] = v

        def fire_gathers(slot):
            for r in range(RB):
                pltpu.make_async_copy(
                    table_hbm.at[idx_v.at[slot].at[r]],
                    rows_v.at[slot].at[r],
                    semg[slot],
                ).start()

        def drain_gathers(slot):
            for r in range(RB):
                pltpu.make_async_copy(
                    table_hbm.at[idx_v.at[slot].at[r]],
                    rows_v.at[slot].at[r],
                    semg[slot],
                ).wait()

        def compute(jc, slot, accs):
            def body(jj, a):
                a = list(a)
                for u in range(UNROLL):
                    j = jj * UNROLL + u
                    wrow = jc * 32 + jj * 2 + u // 4
                    c = (u % 4) * EMB
                    w0 = w_v[wrow, c:c + 16]
                    w1 = w_v[wrow, c + 16:c + 32]
                    for r in range(RB):
                        a[2 * r] = a[2 * r] + rows_v[slot, r, j, 0:16] * w0
                        a[2 * r + 1] = (
                            a[2 * r + 1] + rows_v[slot, r, j, 16:32] * w1)
                return tuple(a)

            return lax.fori_loop(0, JC // UNROLL, body, accs)

        pltpu.sync_copy(blk_src(0, 0), blk_v.at[0])
        transpose_blk(0)
        fire_gathers(0)
        fire_blk(0, 1, 1)

        def gbody(grp, carry):
            zero = jnp.zeros((16,), jnp.float32)
            accs = (zero,) * (2 * RB)
            for jc in range(NJC):
                slot = jc % 2
                nslot = 1 - slot
                if jc < NJC - 1:
                    wait_blk(grp, jc + 1, nslot)
                    transpose_blk(nslot)
                    fire_gathers(nslot)
                else:
                    @pl.when(grp + 1 < NGRP)
                    def _():
                        wait_blk(grp + 1, 0, nslot)
                        transpose_blk(nslot)
                        fire_gathers(nslot)
                if jc < NJC - 2:
                    fire_blk(grp, jc + 2, slot)
                else:
                    @pl.when(grp + 1 < NGRP)
                    def _():
                        fire_blk(grp + 1, jc + 2 - NJC, slot)
                drain_gathers(slot)
                accs = compute(jc, slot, accs)
            for r in range(RB):
                plsc.store_scatter(
                    acc_t_v,
                    [lane * ROWS_PER_W + (grp * RB + r)],
                    accs[2 * r] + accs[2 * r + 1],
                )
            return carry

        lax.fori_loop(0, NGRP, gbody, jnp.int32(0))

        for rc in range(ROWS_PER_W // 16):
            t = acc_t_v[pl.ds(rc * 16, 16)]
            for l in range(1, 16):
                t = t + acc_t_v[pl.ds(l * ROWS_PER_W + rc * 16, 16)]
            out_v[pl.ds(rc * 16, 16)] = t

        pltpu.sync_copy(out_v, out_hbm.at[pl.ds(base, ROWS_PER_W)])

    return run(codes_t, table, w2)


def kernel(codes, table, W, b):
    v = codes.astype(jnp.int32).reshape(BATCH, NUM_LOOKUPS).T
    rho_main = (v & -(4 * TCH)) + ((v & (TCH - 1)) << 2) + ((v >> 9) & 3)
    rt = v - TMAIN
    ta = rt // TTCH
    rho_tail = TMAIN + ((rt - ta * TTCH) << 2) + ta
    codes_t = jnp.where(v < TMAIN, rho_main, rho_tail)
    t2 = _tc_relayout(jnp.swapaxes(table, 0, 1))
    logits = _sc_logits(
        codes_t,
        t2.reshape(CODEBOOK, EMB),
        W.reshape(NUM_LOOKUPS * EMB // 128, 128),
    )
    return logits + b[0]


# speedup vs baseline: 1.2885x; 1.2885x over previous
import functools

import jax
import jax.numpy as jnp
from jax import lax
from jax.experimental import pallas as pl
from jax.experimental.pallas import tpu as pltpu
from jax.experimental.pallas import tpu_sc as plsc

BATCH = 4096
CODEBOOK = 1000000
NUM_LOOKUPS = 1024
EMB = 32
JC = 128
NJC = NUM_LOOKUPS // JC
RB = 8
NC, NS = 2, 16
NW = NC * NS
ROWS_PER_W = BATCH // NW
NGRP = ROWS_PER_W // RB
NSTEP = NGRP * NJC
UNROLL = 8


TCH = 1024
TGRID = CODEBOOK // (4 * TCH)
TMAIN = TGRID * 4 * TCH
TTAIL = CODEBOOK - TMAIN
TTCH = TTAIL // 4
TROWS = CODEBOOK * EMB // 128


def _tc_relayout(t4):
    def main_body(in0, in1, in2, in3, out_ref):
        xs = jnp.concatenate(
            [in0[...], in1[...], in2[...], in3[...]], axis=0)
        out_ref[...] = xs.T

    in_specs = [
        pl.BlockSpec((32, TCH), lambda i, a=a: (0, 4 * i + a))
        for a in range(4)
    ]
    t2 = pl.pallas_call(
        main_body,
        grid=(TGRID,),
        in_specs=in_specs,
        out_specs=pl.BlockSpec((TCH, 128), lambda i: (i, 0)),
        out_shape=jax.ShapeDtypeStruct((TROWS, 128), jnp.float32),
    )(t4, t4, t4, t4)

    t4_tail = lax.slice(t4, (0, TMAIN), (32, CODEBOOK))
    ytail = (t4_tail.reshape(32, 4, TTCH).transpose(1, 0, 2)
             .reshape(128, TTCH).T)
    return lax.dynamic_update_slice(t2, ytail, (TMAIN * EMB // 128, 0))


def _sc_logits(codes_t, table, w2):
    mesh = plsc.VectorSubcoreMesh(core_axis_name="c", subcore_axis_name="s")

    @functools.partial(
        pl.kernel,
        out_type=jax.ShapeDtypeStruct((BATCH,), jnp.float32),
        mesh=mesh,
        compiler_params=pltpu.CompilerParams(
            needs_layout_passes=False, use_tc_tiling_on_sc=False),
        scratch_types=[
            pltpu.VMEM((2, JC, RB), jnp.int32),
            pltpu.VMEM((2, RB, JC), jnp.int32),
            pltpu.VMEM((2, RB, JC, EMB), jnp.float32),
            pltpu.VMEM((NUM_LOOKUPS * EMB // 128, 128), jnp.float32),
            pltpu.VMEM((16 * ROWS_PER_W,), jnp.float32),
            pltpu.VMEM((ROWS_PER_W,), jnp.float32),
            pltpu.SemaphoreType.DMA,
            pltpu.SemaphoreType.DMA,
            pltpu.SemaphoreType.DMA,
            pltpu.SemaphoreType.DMA,
        ],
    )
    def run(codes_hbm, table_hbm, w_hbm, out_hbm,
            blk_v, idx_v, rows_v, w_v, acc_t_v, out_v,
            semc0, semc1, semg0, semg1):
        wid = lax.axis_index("s") * NC + lax.axis_index("c")
        base = wid * ROWS_PER_W
        semc = (semc0, semc1)
        semg = (semg0, semg1)
        lane = lax.iota(jnp.int32, 16)

        pltpu.sync_copy(w_hbm, w_v)

        def blk_src(grp, jc):
            col = base + grp * RB
            return codes_hbm.at[pl.ds(jc * JC, JC), pl.ds(col, RB)]

        def fire_blk(grp, jc, slot):
            pltpu.make_async_copy(blk_src(grp, jc), blk_v.at[slot],
                                  semc[slot]).start()

        def wait_blk(grp, jc, slot):
            pltpu.make_async_copy(blk_src(grp, jc), blk_v.at[slot],
                                  semc[slot]).wait()

        def transpose_blk(slot):
            for r in range(RB):
                rvec = jnp.full((16,), r, jnp.int32)
                for j0 in range(0, JC, 16):
                    v = plsc.load_gather(blk_v.at[slot], [j0 + lane, rvec])
                    idx_v[slot, r, pl.ds(j0, 16)] = v

        def fire_gathers(slot):
            for r in range(RB):
                pltpu.make_async_copy(
                    table_hbm.at[idx_v.at[slot].at[r]],
                    rows_v.at[slot].at[r],
                    semg[slot],
                ).start()

        def drain_gathers(slot):
            for r in range(RB):
                pltpu.make_async_copy(
                    table_hbm.at[idx_v.at[slot].at[r]],
                    rows_v.at[slot].at[r],
                    semg[slot],
                ).wait()

        def compute(jc, slot, accs):
            def body(jj, a):
                a = list(a)
                for u in range(UNROLL):
                    j = jj * UNROLL + u
                    wrow = jc * 32 + jj * 2 + u // 4
                    c = (u % 4) * EMB
                    w0 = w_v[wrow, c:c + 16]
                    w1 = w_v[wrow, c + 16:c + 32]
                    for r in range(RB):
                        a[2 * r] = a[2 * r] + rows_v[slot, r, j, 0:16] * w0
                        a[2 * r + 1] = (
                            a[2 * r + 1] + rows_v[slot, r, j, 16:32] * w1)
                return tuple(a)

            return lax.fori_loop(0, JC // UNROLL, body, accs)

        pltpu.sync_copy(blk_src(0, 0), blk_v.at[0])
        transpose_blk(0)
        fire_gathers(0)
        fire_blk(0, 1, 1)

        def gbody(grp, carry):
            zero = jnp.zeros((16,), jnp.float32)
            accs = (zero,) * (2 * RB)
            for jc in range(NJC):
                slot = jc % 2
                nslot = 1 - slot
                if jc < NJC - 1:
                    wait_blk(grp, jc + 1, nslot)
                    transpose_blk(nslot)
                    fire_gathers(nslot)
                else:
                    @pl.when(grp + 1 < NGRP)
                    def _():
                        wait_blk(grp + 1, 0, nslot)
                        transpose_blk(nslot)
                        fire_gathers(nslot)
                if jc < NJC - 2:
                    fire_blk(grp, jc + 2, slot)
                else:
                    @pl.when(grp + 1 < NGRP)
                    def _():
                        fire_blk(grp + 1, jc + 2 - NJC, slot)
                drain_gathers(slot)
                accs = compute(jc, slot, accs)
            for r in range(RB):
                plsc.store_scatter(
                    acc_t_v,
                    [lane * ROWS_PER_W + (grp * RB + r)],
                    accs[2 * r] + accs[2 * r + 1],
                )
            return carry

        lax.fori_loop(0, NGRP, gbody, jnp.int32(0))

        for rc in range(ROWS_PER_W // 16):
            t = acc_t_v[pl.ds(rc * 16, 16)]
            for l in range(1, 16):
                t = t + acc_t_v[pl.ds(l * ROWS_PER_W + rc * 16, 16)]
            out_v[pl.ds(rc * 16, 16)] = t

        pltpu.sync_copy(out_v, out_hbm.at[pl.ds(base, ROWS_PER_W)])

    return run(codes_t, table, w2)


def kernel(codes, table, W, b):
    v = codes.astype(jnp.int32).reshape(BATCH, NUM_LOOKUPS).T
    rho_main = ((v & -(4 * TCH)) + ((v & (TCH - 1)) << 2)
                + ((v >> (TCH.bit_length() - 1)) & 3))
    rt = v - TMAIN
    ta = rt // TTCH
    rho_tail = TMAIN + ((rt - ta * TTCH) << 2) + ta
    codes_t = jnp.where(v < TMAIN, rho_main, rho_tail)
    t2 = _tc_relayout(jnp.swapaxes(table, 0, 1))
    logits = _sc_logits(
        codes_t,
        t2.reshape(CODEBOOK, EMB),
        W.reshape(NUM_LOOKUPS * EMB // 128, 128),
    )
    return logits + b[0]


# speedup vs baseline: 1.4124x; 1.0962x over previous
import functools

import jax
import jax.numpy as jnp
from jax import lax
from jax.experimental import pallas as pl
from jax.experimental.pallas import tpu as pltpu
from jax.experimental.pallas import tpu_sc as plsc

BATCH = 4096
CODEBOOK = 1000000
NUM_LOOKUPS = 1024
EMB = 32
JC = 128
NJC = NUM_LOOKUPS // JC
RB = 8
NC, NS = 2, 16
NW = NC * NS
ROWS_PER_W = BATCH // NW
NGRP = ROWS_PER_W // RB
NSTEP = NGRP * NJC
UNROLL = 8


TCH = 2048
TGRID = CODEBOOK // (4 * TCH)
TMAIN = TGRID * 4 * TCH
TTAIL = CODEBOOK - TMAIN
TTCH = TTAIL // 4
TROWS = CODEBOOK * EMB // 128


def _tc_relayout(t4):
    def main_body(in0, in1, in2, in3, out_ref):
        xs = jnp.concatenate(
            [in0[...], in1[...], in2[...], in3[...]], axis=0)
        out_ref[...] = xs.T

    in_specs = [
        pl.BlockSpec((32, TCH), lambda i, a=a: (0, 4 * i + a))
        for a in range(4)
    ]
    t2 = pl.pallas_call(
        main_body,
        grid=(TGRID,),
        in_specs=in_specs,
        out_specs=pl.BlockSpec((TCH, 128), lambda i: (i, 0)),
        out_shape=jax.ShapeDtypeStruct((TROWS, 128), jnp.float32),
    )(t4, t4, t4, t4)

    t4_tail = lax.slice(t4, (0, TMAIN), (32, CODEBOOK))
    ytail = (t4_tail.reshape(32, 4, TTCH).transpose(1, 0, 2)
             .reshape(128, TTCH).T)
    return lax.dynamic_update_slice(t2, ytail, (TMAIN * EMB // 128, 0))


def _sc_logits(codes_t, table, w2):
    mesh = plsc.VectorSubcoreMesh(core_axis_name="c", subcore_axis_name="s")

    @functools.partial(
        pl.kernel,
        out_type=jax.ShapeDtypeStruct((BATCH,), jnp.float32),
        mesh=mesh,
        compiler_params=pltpu.CompilerParams(
            needs_layout_passes=False, use_tc_tiling_on_sc=False),
        scratch_types=[
            pltpu.VMEM((2, JC, RB), jnp.int32),
            pltpu.VMEM((2, RB, JC), jnp.int32),
            pltpu.VMEM((2, RB, JC, EMB), jnp.float32),
            pltpu.VMEM((NUM_LOOKUPS * EMB // 128, 128), jnp.float32),
            pltpu.VMEM((16 * ROWS_PER_W,), jnp.float32),
            pltpu.VMEM((ROWS_PER_W,), jnp.float32),
            pltpu.SemaphoreType.DMA,
            pltpu.SemaphoreType.DMA,
            pltpu.SemaphoreType.DMA,
            pltpu.SemaphoreType.DMA,
        ],
    )
    def run(codes_hbm, table_hbm, w_hbm, out_hbm,
            blk_v, idx_v, rows_v, w_v, acc_t_v, out_v,
            semc0, semc1, semg0, semg1):
        wid = lax.axis_index("s") * NC + lax.axis_index("c")
        base = wid * ROWS_PER_W
        semc = (semc0, semc1)
        semg = (semg0, semg1)
        lane = lax.iota(jnp.int32, 16)

        pltpu.sync_copy(w_hbm, w_v)

        def blk_src(grp, jc):
            col = base + grp * RB
            return codes_hbm.at[pl.ds(jc * JC, JC), pl.ds(col, RB)]

        def fire_blk(grp, jc, slot):
            pltpu.make_async_copy(blk_src(grp, jc), blk_v.at[slot],
                                  semc[slot]).start()

        def wait_blk(grp, jc, slot):
            pltpu.make_async_copy(blk_src(grp, jc), blk_v.at[slot],
                                  semc[slot]).wait()

        def transpose_blk(slot):
            for r in range(RB):
                rvec = jnp.full((16,), r, jnp.int32)
                for j0 in range(0, JC, 16):
                    v = plsc.load_gather(blk_v.at[slot], [j0 + lane, rvec])
                    idx_v[slot, r, pl.ds(j0, 16)] = v

        def fire_gathers(slot):
            for r in range(RB):
                pltpu.make_async_copy(
                    table_hbm.at[idx_v.at[slot].at[r]],
                    rows_v.at[slot].at[r],
                    semg[slot],
                ).start()

        def drain_gathers(slot):
            for r in range(RB):
                pltpu.make_async_copy(
                    table_hbm.at[idx_v.at[slot].at[r]],
                    rows_v.at[slot].at[r],
                    semg[slot],
                ).wait()

        def compute(jc, slot, accs):
            def body(jj, a):
                a = list(a)
                for u in range(UNROLL):
                    j = jj * UNROLL + u
                    wrow = jc * 32 + jj * 2 + u // 4
                    c = (u % 4) * EMB
                    w0 = w_v[wrow, c:c + 16]
                    w1 = w_v[wrow, c + 16:c + 32]
                    for r in range(RB):
                        a[2 * r] = a[2 * r] + rows_v[slot, r, j, 0:16] * w0
                        a[2 * r + 1] = (
                            a[2 * r + 1] + rows_v[slot, r, j, 16:32] * w1)
                return tuple(a)

            return lax.fori_loop(0, JC // UNROLL, body, accs)

        pltpu.sync_copy(blk_src(0, 0), blk_v.at[0])
        transpose_blk(0)
        fire_gathers(0)
        fire_blk(0, 1, 1)

        def gbody(grp, carry):
            zero = jnp.zeros((16,), jnp.float32)
            accs = (zero,) * (2 * RB)
            for jc in range(NJC):
                slot = jc % 2
                nslot = 1 - slot
                if jc < NJC - 1:
                    wait_blk(grp, jc + 1, nslot)
                    transpose_blk(nslot)
                    fire_gathers(nslot)
                else:
                    @pl.when(grp + 1 < NGRP)
                    def _():
                        wait_blk(grp + 1, 0, nslot)
                        transpose_blk(nslot)
                        fire_gathers(nslot)
                if jc < NJC - 2:
                    fire_blk(grp, jc + 2, slot)
                else:
                    @pl.when(grp + 1 < NGRP)
                    def _():
                        fire_blk(grp + 1, jc + 2 - NJC, slot)
                drain_gathers(slot)
                accs = compute(jc, slot, accs)
            for r in range(RB):
                plsc.store_scatter(
                    acc_t_v,
                    [lane * ROWS_PER_W + (grp * RB + r)],
                    accs[2 * r] + accs[2 * r + 1],
                )
            return carry

        lax.fori_loop(0, NGRP, gbody, jnp.int32(0))

        for rc in range(ROWS_PER_W // 16):
            t = acc_t_v[pl.ds(rc * 16, 16)]
            for l in range(1, 16):
                t = t + acc_t_v[pl.ds(l * ROWS_PER_W + rc * 16, 16)]
            out_v[pl.ds(rc * 16, 16)] = t

        pltpu.sync_copy(out_v, out_hbm.at[pl.ds(base, ROWS_PER_W)])

    return run(codes_t, table, w2)


def kernel(codes, table, W, b):
    v = codes.astype(jnp.int32).reshape(BATCH, NUM_LOOKUPS).T
    rho_main = ((v & -(4 * TCH)) + ((v & (TCH - 1)) << 2)
                + ((v >> (TCH.bit_length() - 1)) & 3))
    rt = v - TMAIN
    ta = rt // TTCH
    rho_tail = TMAIN + ((rt - ta * TTCH) << 2) + ta
    codes_t = jnp.where(v < TMAIN, rho_main, rho_tail)
    t2 = _tc_relayout(jnp.swapaxes(table, 0, 1))
    logits = _sc_logits(
        codes_t,
        t2.reshape(CODEBOOK, EMB),
        W.reshape(NUM_LOOKUPS * EMB // 128, 128),
    )
    return logits + b[0]


# speedup vs baseline: 1.5072x; 1.0671x over previous
import functools

import jax
import jax.numpy as jnp
from jax import lax
from jax.experimental import pallas as pl
from jax.experimental.pallas import tpu as pltpu
from jax.experimental.pallas import tpu_sc as plsc

BATCH = 4096
CODEBOOK = 1000000
NUM_LOOKUPS = 1024
EMB = 32
JC = 128
NJC = NUM_LOOKUPS // JC
RB = 8
NC, NS = 2, 16
NW = NC * NS
ROWS_PER_W = BATCH // NW
NGRP = ROWS_PER_W // RB
NSTEP = NGRP * NJC
UNROLL = 8


TCH = 4096
TGRID = CODEBOOK // (4 * TCH)
TMAIN = TGRID * 4 * TCH
TTAIL = CODEBOOK - TMAIN
TTCH = TTAIL // 4
TROWS = CODEBOOK * EMB // 128


def _tc_relayout(t4):
    def main_body(in0, in1, in2, in3, out_ref):
        xs = jnp.concatenate(
            [in0[...], in1[...], in2[...], in3[...]], axis=0)
        out_ref[...] = xs.T

    in_specs = [
        pl.BlockSpec((32, TCH), lambda i, a=a: (0, 4 * i + a))
        for a in range(4)
    ]
    t2 = pl.pallas_call(
        main_body,
        grid=(TGRID,),
        in_specs=in_specs,
        out_specs=pl.BlockSpec((TCH, 128), lambda i: (i, 0)),
        out_shape=jax.ShapeDtypeStruct((TROWS, 128), jnp.float32),
    )(t4, t4, t4, t4)

    t4_tail = lax.slice(t4, (0, TMAIN), (32, CODEBOOK))
    ytail = (t4_tail.reshape(32, 4, TTCH).transpose(1, 0, 2)
             .reshape(128, TTCH).T)
    return lax.dynamic_update_slice(t2, ytail, (TMAIN * EMB // 128, 0))


def _sc_logits(codes_t, table, w2):
    mesh = plsc.VectorSubcoreMesh(core_axis_name="c", subcore_axis_name="s")

    @functools.partial(
        pl.kernel,
        out_type=jax.ShapeDtypeStruct((BATCH,), jnp.float32),
        mesh=mesh,
        compiler_params=pltpu.CompilerParams(
            needs_layout_passes=False, use_tc_tiling_on_sc=False),
        scratch_types=[
            pltpu.VMEM((2, JC, RB), jnp.int32),
            pltpu.VMEM((2, RB, JC), jnp.int32),
            pltpu.VMEM((2, RB, JC, EMB), jnp.float32),
            pltpu.VMEM((NUM_LOOKUPS * EMB // 128, 128), jnp.float32),
            pltpu.VMEM((16 * ROWS_PER_W,), jnp.float32),
            pltpu.VMEM((ROWS_PER_W,), jnp.float32),
            pltpu.SemaphoreType.DMA,
            pltpu.SemaphoreType.DMA,
            pltpu.SemaphoreType.DMA,
            pltpu.SemaphoreType.DMA,
        ],
    )
    def run(codes_hbm, table_hbm, w_hbm, out_hbm,
            blk_v, idx_v, rows_v, w_v, acc_t_v, out_v,
            semc0, semc1, semg0, semg1):
        wid = lax.axis_index("s") * NC + lax.axis_index("c")
        base = wid * ROWS_PER_W
        semc = (semc0, semc1)
        semg = (semg0, semg1)
        lane = lax.iota(jnp.int32, 16)

        pltpu.sync_copy(w_hbm, w_v)

        def blk_src(grp, jc):
            col = base + grp * RB
            return codes_hbm.at[pl.ds(jc * JC, JC), pl.ds(col, RB)]

        def fire_blk(grp, jc, slot):
            pltpu.make_async_copy(blk_src(grp, jc), blk_v.at[slot],
                                  semc[slot]).start()

        def wait_blk(grp, jc, slot):
            pltpu.make_async_copy(blk_src(grp, jc), blk_v.at[slot],
                                  semc[slot]).wait()

        def transpose_blk(slot):
            for r in range(RB):
                rvec = jnp.full((16,), r, jnp.int32)
                for j0 in range(0, JC, 16):
                    v = plsc.load_gather(blk_v.at[slot], [j0 + lane, rvec])
                    idx_v[slot, r, pl.ds(j0, 16)] = v

        def fire_gathers(slot):
            for r in range(RB):
                pltpu.make_async_copy(
                    table_hbm.at[idx_v.at[slot].at[r]],
                    rows_v.at[slot].at[r],
                    semg[slot],
                ).start()

        def drain_gathers(slot):
            for r in range(RB):
                pltpu.make_async_copy(
                    table_hbm.at[idx_v.at[slot].at[r]],
                    rows_v.at[slot].at[r],
                    semg[slot],
                ).wait()

        def compute(jc, slot, accs):
            def body(jj, a):
                a = list(a)
                for u in range(UNROLL):
                    j = jj * UNROLL + u
                    wrow = jc * 32 + jj * 2 + u // 4
                    c = (u % 4) * EMB
                    w0 = w_v[wrow, c:c + 16]
                    w1 = w_v[wrow, c + 16:c + 32]
                    for r in range(RB):
                        a[2 * r] = a[2 * r] + rows_v[slot, r, j, 0:16] * w0
                        a[2 * r + 1] = (
                            a[2 * r + 1] + rows_v[slot, r, j, 16:32] * w1)
                return tuple(a)

            return lax.fori_loop(0, JC // UNROLL, body, accs)

        pltpu.sync_copy(blk_src(0, 0), blk_v.at[0])
        transpose_blk(0)
        fire_gathers(0)
        fire_blk(0, 1, 1)

        def gbody(grp, carry):
            zero = jnp.zeros((16,), jnp.float32)
            accs = (zero,) * (2 * RB)
            for jc in range(NJC):
                slot = jc % 2
                nslot = 1 - slot
                if jc < NJC - 1:
                    wait_blk(grp, jc + 1, nslot)
                    transpose_blk(nslot)
                    fire_gathers(nslot)
                else:
                    @pl.when(grp + 1 < NGRP)
                    def _():
                        wait_blk(grp + 1, 0, nslot)
                        transpose_blk(nslot)
                        fire_gathers(nslot)
                if jc < NJC - 2:
                    fire_blk(grp, jc + 2, slot)
                else:
                    @pl.when(grp + 1 < NGRP)
                    def _():
                        fire_blk(grp + 1, jc + 2 - NJC, slot)
                drain_gathers(slot)
                accs = compute(jc, slot, accs)
            for r in range(RB):
                plsc.store_scatter(
                    acc_t_v,
                    [lane * ROWS_PER_W + (grp * RB + r)],
                    accs[2 * r] + accs[2 * r + 1],
                )
            return carry

        lax.fori_loop(0, NGRP, gbody, jnp.int32(0))

        for rc in range(ROWS_PER_W // 16):
            t = acc_t_v[pl.ds(rc * 16, 16)]
            for l in range(1, 16):
                t = t + acc_t_v[pl.ds(l * ROWS_PER_W + rc * 16, 16)]
            out_v[pl.ds(rc * 16, 16)] = t

        pltpu.sync_copy(out_v, out_hbm.at[pl.ds(base, ROWS_PER_W)])

    return run(codes_t, table, w2)


def kernel(codes, table, W, b):
    v = codes.astype(jnp.int32).reshape(BATCH, NUM_LOOKUPS).T
    rho_main = ((v & -(4 * TCH)) + ((v & (TCH - 1)) << 2)
                + ((v >> (TCH.bit_length() - 1)) & 3))
    rt = v - TMAIN
    ta = rt // TTCH
    rho_tail = TMAIN + ((rt - ta * TTCH) << 2) + ta
    codes_t = jnp.where(v < TMAIN, rho_main, rho_tail)
    t2 = _tc_relayout(jnp.swapaxes(table, 0, 1))
    logits = _sc_logits(
        codes_t,
        t2.reshape(CODEBOOK, EMB),
        W.reshape(NUM_LOOKUPS * EMB // 128, 128),
    )
    return logits + b[0]


# speedup vs baseline: 1.5294x; 1.0147x over previous
import functools

import jax
import jax.numpy as jnp
from jax import lax
from jax.experimental import pallas as pl
from jax.experimental.pallas import tpu as pltpu
from jax.experimental.pallas import tpu_sc as plsc

BATCH = 4096
CODEBOOK = 1000000
NUM_LOOKUPS = 1024
EMB = 32
JC = 128
NJC = NUM_LOOKUPS // JC
RB = 8
NC, NS = 2, 16
NW = NC * NS
ROWS_PER_W = BATCH // NW
NGRP = ROWS_PER_W // RB
NSTEP = NGRP * NJC
UNROLL = 8


TCH = 8192
TGRID = CODEBOOK // (4 * TCH)
TMAIN = TGRID * 4 * TCH
TTAIL = CODEBOOK - TMAIN
TTCH = TTAIL // 4
TROWS = CODEBOOK * EMB // 128


def _tc_relayout(t4):
    def main_body(in0, in1, in2, in3, out_ref):
        xs = jnp.concatenate(
            [in0[...], in1[...], in2[...], in3[...]], axis=0)
        out_ref[...] = xs.T

    in_specs = [
        pl.BlockSpec((32, TCH), lambda i, a=a: (0, 4 * i + a))
        for a in range(4)
    ]
    t2 = pl.pallas_call(
        main_body,
        grid=(TGRID,),
        in_specs=in_specs,
        out_specs=pl.BlockSpec((TCH, 128), lambda i: (i, 0)),
        out_shape=jax.ShapeDtypeStruct((TROWS, 128), jnp.float32),
    )(t4, t4, t4, t4)

    t4_tail = lax.slice(t4, (0, TMAIN), (32, CODEBOOK))
    ytail = (t4_tail.reshape(32, 4, TTCH).transpose(1, 0, 2)
             .reshape(128, TTCH).T)
    return lax.dynamic_update_slice(t2, ytail, (TMAIN * EMB // 128, 0))


def _sc_logits(codes_t, table, w2):
    mesh = plsc.VectorSubcoreMesh(core_axis_name="c", subcore_axis_name="s")

    @functools.partial(
        pl.kernel,
        out_type=jax.ShapeDtypeStruct((BATCH,), jnp.float32),
        mesh=mesh,
        compiler_params=pltpu.CompilerParams(
            needs_layout_passes=False, use_tc_tiling_on_sc=False),
        scratch_types=[
            pltpu.VMEM((2, JC, RB), jnp.int32),
            pltpu.VMEM((2, RB, JC), jnp.int32),
            pltpu.VMEM((2, RB, JC, EMB), jnp.float32),
            pltpu.VMEM((NUM_LOOKUPS * EMB // 128, 128), jnp.float32),
            pltpu.VMEM((16 * ROWS_PER_W,), jnp.float32),
            pltpu.VMEM((ROWS_PER_W,), jnp.float32),
            pltpu.SemaphoreType.DMA,
            pltpu.SemaphoreType.DMA,
            pltpu.SemaphoreType.DMA,
            pltpu.SemaphoreType.DMA,
        ],
    )
    def run(codes_hbm, table_hbm, w_hbm, out_hbm,
            blk_v, idx_v, rows_v, w_v, acc_t_v, out_v,
            semc0, semc1, semg0, semg1):
        wid = lax.axis_index("s") * NC + lax.axis_index("c")
        base = wid * ROWS_PER_W
        semc = (semc0, semc1)
        semg = (semg0, semg1)
        lane = lax.iota(jnp.int32, 16)

        pltpu.sync_copy(w_hbm, w_v)

        def blk_src(grp, jc):
            col = base + grp * RB
            return codes_hbm.at[pl.ds(jc * JC, JC), pl.ds(col, RB)]

        def fire_blk(grp, jc, slot):
            pltpu.make_async_copy(blk_src(grp, jc), blk_v.at[slot],
                                  semc[slot]).start()

        def wait_blk(grp, jc, slot):
            pltpu.make_async_copy(blk_src(grp, jc), blk_v.at[slot],
                                  semc[slot]).wait()

        def transpose_blk(slot):
            for r in range(RB):
                rvec = jnp.full((16,), r, jnp.int32)
                for j0 in range(0, JC, 16):
                    v = plsc.load_gather(blk_v.at[slot], [j0 + lane, rvec])
                    idx_v[slot, r, pl.ds(j0, 16)] = v

        def fire_gathers(slot):
            for r in range(RB):
                pltpu.make_async_copy(
                    table_hbm.at[idx_v.at[slot].at[r]],
                    rows_v.at[slot].at[r],
                    semg[slot],
                ).start()

        def drain_gathers(slot):
            for r in range(RB):
                pltpu.make_async_copy(
                    table_hbm.at[idx_v.at[slot].at[r]],
                    rows_v.at[slot].at[r],
                    semg[slot],
                ).wait()

        def compute(jc, slot, accs):
            def body(jj, a):
                a = list(a)
                for u in range(UNROLL):
                    j = jj * UNROLL + u
                    wrow = jc * 32 + jj * 2 + u // 4
                    c = (u % 4) * EMB
                    w0 = w_v[wrow, c:c + 16]
                    w1 = w_v[wrow, c + 16:c + 32]
                    for r in range(RB):
                        a[2 * r] = a[2 * r] + rows_v[slot, r, j, 0:16] * w0
                        a[2 * r + 1] = (
                            a[2 * r + 1] + rows_v[slot, r, j, 16:32] * w1)
                return tuple(a)

            return lax.fori_loop(0, JC // UNROLL, body, accs)

        pltpu.sync_copy(blk_src(0, 0), blk_v.at[0])
        transpose_blk(0)
        fire_gathers(0)
        fire_blk(0, 1, 1)

        def gbody(grp, carry):
            zero = jnp.zeros((16,), jnp.float32)
            accs = (zero,) * (2 * RB)
            for jc in range(NJC):
                slot = jc % 2
                nslot = 1 - slot
                if jc < NJC - 1:
                    wait_blk(grp, jc + 1, nslot)
                    transpose_blk(nslot)
                    fire_gathers(nslot)
                else:
                    @pl.when(grp + 1 < NGRP)
                    def _():
                        wait_blk(grp + 1, 0, nslot)
                        transpose_blk(nslot)
                        fire_gathers(nslot)
                if jc < NJC - 2:
                    fire_blk(grp, jc + 2, slot)
                else:
                    @pl.when(grp + 1 < NGRP)
                    def _():
                        fire_blk(grp + 1, jc + 2 - NJC, slot)
                drain_gathers(slot)
                accs = compute(jc, slot, accs)
            for r in range(RB):
                plsc.store_scatter(
                    acc_t_v,
                    [lane * ROWS_PER_W + (grp * RB + r)],
                    accs[2 * r] + accs[2 * r + 1],
                )
            return carry

        lax.fori_loop(0, NGRP, gbody, jnp.int32(0))

        for rc in range(ROWS_PER_W // 16):
            t = acc_t_v[pl.ds(rc * 16, 16)]
            for l in range(1, 16):
                t = t + acc_t_v[pl.ds(l * ROWS_PER_W + rc * 16, 16)]
            out_v[pl.ds(rc * 16, 16)] = t

        pltpu.sync_copy(out_v, out_hbm.at[pl.ds(base, ROWS_PER_W)])

    return run(codes_t, table, w2)


def kernel(codes, table, W, b):
    v = codes.astype(jnp.int32).reshape(BATCH, NUM_LOOKUPS).T
    rho_main = ((v & -(4 * TCH)) + ((v & (TCH - 1)) << 2)
                + ((v >> (TCH.bit_length() - 1)) & 3))
    rt = v - TMAIN
    ta = rt // TTCH
    rho_tail = TMAIN + ((rt - ta * TTCH) << 2) + ta
    codes_t = jnp.where(v < TMAIN, rho_main, rho_tail)
    t2 = _tc_relayout(jnp.swapaxes(table, 0, 1))
    logits = _sc_logits(
        codes_t,
        t2.reshape(CODEBOOK, EMB),
        W.reshape(NUM_LOOKUPS * EMB // 128, 128),
    )
    return logits + b[0]
